# Initial kernel scaffold; baseline (speedup 1.0000x reference)
import functools

import jax
import jax.numpy as jnp
from jax import lax
from jax.experimental import pallas as pl
from jax.experimental.pallas import tpu as pltpu
from jax.experimental.pallas import tpu_sc as plsc


def _gather_kernel(B, D, b_per_w, chunk, n_chunks, NC):
    mesh = plsc.VectorSubcoreMesh(core_axis_name="c", subcore_axis_name="s")

    @functools.partial(
        pl.kernel,
        mesh=mesh,
        out_type=jax.ShapeDtypeStruct((B, D), jnp.float32),
        scratch_types=[
            pltpu.VMEM((chunk,), jnp.int32),
            pltpu.VMEM((chunk, D), jnp.float32),
            pltpu.SemaphoreType.DMA,
        ],
    )
    def k(idx_hbm, table_hbm, out_hbm, idx_v, rows_v, sem):
        wid = lax.axis_index("s") * NC + lax.axis_index("c")
        base = wid * b_per_w

        def body(i, carry):
            off = base + i * chunk
            pltpu.sync_copy(idx_hbm.at[pl.ds(off, chunk)], idx_v)
            pltpu.async_copy(table_hbm.at[idx_v], rows_v, sem).wait()
            pltpu.sync_copy(rows_v, out_hbm.at[pl.ds(off, chunk)])
            return carry

        lax.fori_loop(0, n_chunks, body, 0)

    return k


def kernel(token_ids, weight):
    B0, B1 = token_ids.shape
    V, D = weight.shape
    B = B0 * B1

    info = plsc.get_sparse_core_info()
    NC, NS = info.num_cores, info.num_subcores
    NW = NC * NS
    b_per_w = B // NW
    chunk = 1024
    n_chunks = b_per_w // chunk

    idx = token_ids.reshape(B).astype(jnp.int32)
    out = _gather_kernel(B, D, b_per_w, chunk, n_chunks, NC)(idx, weight)
    return out.reshape(B0, B1, D)


# speedup vs baseline: 1.0932x; 1.0932x over previous
import functools

import jax
import jax.numpy as jnp
from jax import lax
from jax.experimental import pallas as pl
from jax.experimental.pallas import tpu as pltpu
from jax.experimental.pallas import tpu_sc as plsc


def _gather_kernel(B, D, b_per_w, chunk, n_chunks, NC):
    mesh = plsc.VectorSubcoreMesh(core_axis_name="c", subcore_axis_name="s")

    @functools.partial(
        pl.kernel,
        mesh=mesh,
        out_type=jax.ShapeDtypeStruct((B, D), jnp.float32),
        scratch_types=[
            pltpu.VMEM((chunk,), jnp.int32),
            pltpu.VMEM((chunk, D), jnp.float32),
            pltpu.SemaphoreType.DMA,
        ],
        compiler_params=pltpu.CompilerParams(use_tc_tiling_on_sc=False),
    )
    def k(idx_hbm, table_hbm, out_hbm, idx_v, rows_v, sem):
        wid = lax.axis_index("s") * NC + lax.axis_index("c")
        base = wid * b_per_w

        def body(i, carry):
            off = base + i * chunk
            pltpu.sync_copy(idx_hbm.at[pl.ds(off, chunk)], idx_v)
            pltpu.async_copy(table_hbm.at[idx_v], rows_v, sem).wait()
            pltpu.sync_copy(rows_v, out_hbm.at[pl.ds(off, chunk)])
            return carry

        lax.fori_loop(0, n_chunks, body, 0)

    return k


def kernel(token_ids, weight):
    B0, B1 = token_ids.shape
    V, D = weight.shape
    B = B0 * B1

    info = plsc.get_sparse_core_info()
    NC, NS = info.num_cores, info.num_subcores
    NW = NC * NS
    b_per_w = B // NW
    chunk = 1024
    n_chunks = b_per_w // chunk

    idx = token_ids.reshape(B).astype(jnp.int32)
    out = _gather_kernel(B, D, b_per_w, chunk, n_chunks, NC)(idx, weight)
    return out.reshape(B0, B1, D)


# speedup vs baseline: 1.4542x; 1.3303x over previous
import functools

import jax
import jax.numpy as jnp
from jax import lax
from jax.experimental import pallas as pl
from jax.experimental.pallas import tpu as pltpu
from jax.experimental.pallas import tpu_sc as plsc

_CB = 128
_SG = 2
_NG = 25
_NK = 4


def _gather_kernel(B0, B1, D, NC):
    mesh = plsc.VectorSubcoreMesh(core_axis_name="c", subcore_axis_name="s")
    rows_per_group = _SG * _CB
    obuf_len = _SG * D * _CB

    @functools.partial(
        pl.kernel,
        mesh=mesh,
        out_type=jax.ShapeDtypeStruct((B1, 4, B0 // _CB, 1024), jnp.float32),
        scratch_types=[
            pltpu.VMEM((_CB * B1,), jnp.int32),
            pltpu.VMEM((rows_per_group,), jnp.int32),
            pltpu.VMEM((rows_per_group, D), jnp.float32),
            pltpu.VMEM((obuf_len,), jnp.float32),
            pltpu.SemaphoreType.DMA,
            pltpu.SemaphoreType.DMA,
        ],
        compiler_params=pltpu.CompilerParams(
            use_tc_tiling_on_sc=False, needs_layout_passes=False
        ),
    )
    def k(idx_hbm, table_hbm, out_hbm, idxs_v, subidx_v, rows_v, obuf_v,
          gat_sem, out_sem):
        wid = lax.axis_index("s") * NC + lax.axis_index("c")
        b_base = wid * (_NK * _CB)
        lanes = lax.iota(jnp.int32, 16)
        lanes50 = lanes * B1
        lanes32 = lanes * D

        def group(kg, carry):
            kk = kg // _NG
            g = lax.rem(kg, _NG)
            b0 = b_base + kk * _CB
            bhi = wid * _NK + kk
            s0 = g * _SG

            @pl.when(g == 0)
            def _():
                pltpu.sync_copy(idx_hbm.at[pl.ds(b0 * B1, _CB * B1)], idxs_v)

            def build(i2, c2):
                s_l = i2 // 8
                bl0 = lax.rem(i2, 8) * 16
                pos = lanes50 + (bl0 * B1 + s0 + s_l)
                v = plsc.load_gather(idxs_v, [pos])
                subidx_v[pl.ds(i2 * 16, 16)] = v
                return c2

            lax.fori_loop(0, rows_per_group // 16, build, 0)

            pltpu.async_copy(table_hbm.at[subidx_v], rows_v, gat_sem).wait()

            def trans(i3, c3):
                s_l = i3 // D
                c = lax.rem(i3, D)
                cols = jnp.full((16,), c, jnp.int32)
                robase = s_l * _CB
                obase = s_l * 4096 + (c // 8) * 1024 + lax.rem(c, 8) * 128
                for j2 in range(_CB // 16):
                    rpos = lanes + (robase + j2 * 16)
                    v = plsc.load_gather(rows_v, [rpos, cols])
                    obuf_v[pl.ds(obase + j2 * 16, 16)] = v
                return c3

            lax.fori_loop(0, _SG * D, trans, 0)

            def emit(i4, c4):
                s_l = i4 // 4
                chi = lax.rem(i4, 4)
                pltpu.async_copy(
                    obuf_v.at[pl.ds(s_l * 4096 + chi * 1024, 1024)],
                    out_hbm.at[s0 + s_l, chi, bhi],
                    out_sem,
                )
                return c4

            lax.fori_loop(0, _SG * 4, emit, 0)

            def drain(i5, c5):
                pltpu.make_async_copy(
                    obuf_v.at[pl.ds(0, 1024)],
                    out_hbm.at[0, 0, 0], out_sem,
                ).wait()
                return c5

            lax.fori_loop(0, _SG * 4, drain, 0)
            return carry

        lax.fori_loop(0, _NK * _NG, group, 0)

    return k


def kernel(token_ids, weight):
    B0, B1 = token_ids.shape
    V, D = weight.shape

    info = plsc.get_sparse_core_info()
    NC = info.num_cores

    idx = token_ids.reshape(B0 * B1).astype(jnp.int32)
    lin = _gather_kernel(B0, B1, D, NC)(idx, weight)
    lin5 = lin.reshape(B1, 4, B0 // _CB, 8, _CB)
    return lin5.transpose((2, 4, 0, 1, 3)).reshape(B0, B1, D)


# speedup vs baseline: 1.4546x; 1.0003x over previous
import functools

import jax
import jax.numpy as jnp
from jax import lax
from jax.experimental import pallas as pl
from jax.experimental.pallas import tpu as pltpu
from jax.experimental.pallas import tpu_sc as plsc

_CB = 128
_SG = 2
_NG = 25
_NK = 4


def _gather_kernel(B0, B1, D, NC):
    mesh = plsc.VectorSubcoreMesh(core_axis_name="c", subcore_axis_name="s")
    rows_per_group = _SG * _CB
    obuf_len = _SG * D * _CB

    @functools.partial(
        pl.kernel,
        mesh=mesh,
        out_type=jax.ShapeDtypeStruct((B1, 4, B0 // _CB, 1024), jnp.float32),
        scratch_types=[
            pltpu.VMEM((_CB * B1,), jnp.int32),
            pltpu.VMEM((rows_per_group,), jnp.int32),
            pltpu.VMEM((rows_per_group, D), jnp.float32),
            pltpu.VMEM((obuf_len,), jnp.float32),
            pltpu.SemaphoreType.DMA,
            pltpu.SemaphoreType.DMA,
        ],
        compiler_params=pltpu.CompilerParams(
            use_tc_tiling_on_sc=False, needs_layout_passes=False
        ),
    )
    def k(idx_hbm, table_hbm, out_hbm, idxs_v, subidx_v, rows_v, obuf_v,
          gat_sem, out_sem):
        wid = lax.axis_index("s") * NC + lax.axis_index("c")
        b_base = wid * (_NK * _CB)
        lanes = lax.iota(jnp.int32, 16)
        lanes50 = lanes * B1
        lanes32 = lanes * D

        def group(kg, carry):
            kk = kg // _NG
            g = lax.rem(kg, _NG)
            b0 = b_base + kk * _CB
            bhi = wid * _NK + kk
            s0 = g * _SG

            @pl.when(g == 0)
            def _():
                pltpu.sync_copy(idx_hbm.at[pl.ds(b0 * B1, _CB * B1)], idxs_v)

            def build(i2, c2):
                s_l = i2 // 8
                bl0 = lax.rem(i2, 8) * 16
                pos = lanes50 + (bl0 * B1 + s0 + s_l)
                v = plsc.load_gather(idxs_v, [pos])
                subidx_v[pl.ds(i2 * 16, 16)] = v
                return c2

            lax.fori_loop(0, rows_per_group // 16, build, 0, unroll=4)

            pltpu.async_copy(table_hbm.at[subidx_v], rows_v, gat_sem).wait()

            def trans(i3, c3):
                s_l = i3 // D
                c = lax.rem(i3, D)
                cols = jnp.full((16,), c, jnp.int32)
                robase = s_l * _CB
                obase = s_l * 4096 + (c // 8) * 1024 + lax.rem(c, 8) * 128
                for j2 in range(_CB // 16):
                    rpos = lanes + (robase + j2 * 16)
                    v = plsc.load_gather(rows_v, [rpos, cols])
                    obuf_v[pl.ds(obase + j2 * 16, 16)] = v
                return c3

            lax.fori_loop(0, _SG * D, trans, 0, unroll=4)

            def emit(i4, c4):
                s_l = i4 // 4
                chi = lax.rem(i4, 4)
                pltpu.async_copy(
                    obuf_v.at[pl.ds(s_l * 4096 + chi * 1024, 1024)],
                    out_hbm.at[s0 + s_l, chi, bhi],
                    out_sem,
                )
                return c4

            lax.fori_loop(0, _SG * 4, emit, 0)

            def drain(i5, c5):
                pltpu.make_async_copy(
                    obuf_v.at[pl.ds(0, 1024)],
                    out_hbm.at[0, 0, 0], out_sem,
                ).wait()
                return c5

            lax.fori_loop(0, _SG * 4, drain, 0)
            return carry

        lax.fori_loop(0, _NK * _NG, group, 0)

    return k


def kernel(token_ids, weight):
    B0, B1 = token_ids.shape
    V, D = weight.shape

    info = plsc.get_sparse_core_info()
    NC = info.num_cores

    idx = token_ids.reshape(B0 * B1).astype(jnp.int32)
    lin = _gather_kernel(B0, B1, D, NC)(idx, weight)
    lin5 = lin.reshape(B1, 4, B0 // _CB, 8, _CB)
    return lin5.transpose((2, 4, 0, 1, 3)).reshape(B0, B1, D)


# speedup vs baseline: 1.5853x; 1.0899x over previous
import functools

import jax
import jax.numpy as jnp
from jax import lax
from jax.experimental import pallas as pl
from jax.experimental.pallas import tpu as pltpu
from jax.experimental.pallas import tpu_sc as plsc

_CB = 128
_SG = 2
_NG = 25
_NK = 4


def _gather_kernel(B0, B1, D, NC):
    mesh = plsc.VectorSubcoreMesh(core_axis_name="c", subcore_axis_name="s")
    rpg = _SG * _CB
    obuf_len = _SG * D * _CB

    @functools.partial(
        pl.kernel,
        mesh=mesh,
        out_type=jax.ShapeDtypeStruct((B1, 4, B0 // _CB, 1024), jnp.float32),
        scratch_types=[
            pltpu.VMEM((_CB * B1,), jnp.int32),
            pltpu.VMEM((2, rpg), jnp.int32),
            pltpu.VMEM((2, rpg, D), jnp.float32),
            pltpu.VMEM((obuf_len,), jnp.float32),
            pltpu.SemaphoreType.DMA,
            pltpu.SemaphoreType.DMA,
            pltpu.SemaphoreType.DMA,
        ],
        compiler_params=pltpu.CompilerParams(
            use_tc_tiling_on_sc=False, needs_layout_passes=False
        ),
    )
    def k(idx_hbm, table_hbm, out_hbm, idxs_v, subidx_v, rows_v, obuf_v,
          gsem0, gsem1, out_sem):
        wid = lax.axis_index("s") * NC + lax.axis_index("c")
        b_base = wid * (_NK * _CB)
        lanes = lax.iota(jnp.int32, 16)
        lanes50 = lanes * B1

        def build(g, slot):
            s0 = g * _SG

            def body(i2, c2):
                s_l = i2 // 8
                bl0 = lax.rem(i2, 8) * 16
                pos = lanes50 + (bl0 * B1 + s0 + s_l)
                subidx_v[slot, pl.ds(i2 * 16, 16)] = plsc.load_gather(
                    idxs_v, [pos]
                )
                return c2

            lax.fori_loop(0, rpg // 16, body, 0)

        def gather_start(slot):
            @pl.when(slot == 0)
            def _():
                pltpu.async_copy(
                    table_hbm.at[subidx_v.at[0]], rows_v.at[0], gsem0
                )

            @pl.when(slot == 1)
            def _():
                pltpu.async_copy(
                    table_hbm.at[subidx_v.at[1]], rows_v.at[1], gsem1
                )

        def gather_wait(slot):
            @pl.when(slot == 0)
            def _():
                pltpu.make_async_copy(
                    table_hbm.at[subidx_v.at[0]], rows_v.at[0], gsem0
                ).wait()

            @pl.when(slot == 1)
            def _():
                pltpu.make_async_copy(
                    table_hbm.at[subidx_v.at[1]], rows_v.at[1], gsem1
                ).wait()

        def drain_emits():
            def body(i5, c5):
                pltpu.make_async_copy(
                    obuf_v.at[pl.ds(0, 1024)], out_hbm.at[0, 0, 0], out_sem
                ).wait()
                return c5

            lax.fori_loop(0, _SG * 4, body, 0)

        for kk in range(_NK):
            b0 = b_base + kk * _CB
            bhi = wid * _NK + kk
            pltpu.sync_copy(idx_hbm.at[pl.ds(b0 * B1, _CB * B1)], idxs_v)
            build(0, 0)
            gather_start(0)

            def group(g, carry):
                p = lax.rem(g, 2)
                s0 = g * _SG

                @pl.when(g + 1 < _NG)
                def _():
                    build(g + 1, 1 - p)
                    gather_start(1 - p)

                gather_wait(p)

                @pl.when(g >= 1)
                def _():
                    drain_emits()

                def trans(i3, c3):
                    s_l = i3 // D
                    c = lax.rem(i3, D)
                    cols = jnp.full((16,), c, jnp.int32)
                    robase = s_l * _CB
                    obase = (s_l * 4096 + (c // 8) * 1024
                             + lax.rem(c, 8) * 128)
                    for j2 in range(_CB // 16):
                        rpos = lanes + (robase + j2 * 16)
                        v = plsc.load_gather(rows_v.at[p], [rpos, cols])
                        obuf_v[pl.ds(obase + j2 * 16, 16)] = v
                    return c3

                lax.fori_loop(0, _SG * D, trans, 0)

                def emit(i4, c4):
                    s_l = i4 // 4
                    chi = lax.rem(i4, 4)
                    pltpu.async_copy(
                        obuf_v.at[pl.ds(s_l * 4096 + chi * 1024, 1024)],
                        out_hbm.at[s0 + s_l, chi, bhi],
                        out_sem,
                    )
                    return c4

                lax.fori_loop(0, _SG * 4, emit, 0)
                return carry

            lax.fori_loop(0, _NG, group, 0)
            drain_emits()

    return k


def kernel(token_ids, weight):
    B0, B1 = token_ids.shape
    V, D = weight.shape

    info = plsc.get_sparse_core_info()
    NC = info.num_cores

    idx = token_ids.reshape(B0 * B1).astype(jnp.int32)
    lin = _gather_kernel(B0, B1, D, NC)(idx, weight)
    lin5 = lin.reshape(B1, 4, B0 // _CB, 8, _CB)
    return lin5.transpose((2, 4, 0, 1, 3)).reshape(B0, B1, D)
